# baseline (speedup 1.0000x reference)
import math

import jax
import jax.numpy as jnp
from jax.experimental import pallas as pl
from jax.experimental.pallas import tpu as pltpu

_SNR_W = 0.5
_ENERGY_W = 0.3
_RECENCY_C = 0.2 * math.exp(-0.1)

_N_TOTAL = 4 * 32 * 128 * 128
_ROWS = _N_TOTAL // 128
_CH = 2048
_NCHUNK = _ROWS // _CH
_CHO = 512
_NCHUNKO = _ROWS // _CHO
_LOG2E = math.log2(math.e)


def _fused_kernel(x_hbm, o_hbm, x_vmem, sem_in, sem_out):
    for i in range(_NCHUNK):
        pltpu.make_async_copy(
            x_hbm.at[pl.ds(i * _CH, _CH)],
            x_vmem.at[pl.ds(i * _CH, _CH)],
            sem_in.at[i],
        ).start()

    def p1(g, carry):
        s, ss, m = carry
        pltpu.make_async_copy(
            x_hbm.at[pl.ds(g * _CH, _CH)],
            x_vmem.at[pl.ds(g * _CH, _CH)],
            sem_in.at[g],
        ).wait()
        x = x_vmem[pl.ds(g * _CH, _CH), :].reshape(_CH // 8, 8, 128)
        s = s + jnp.sum(x, axis=0)
        ss = ss + jnp.sum(x * x, axis=0)
        m = jnp.maximum(m, jnp.max(jnp.abs(x), axis=0))
        return s, ss, m

    z = jnp.zeros((8, 128), jnp.float32)
    s, ss, m = jax.lax.fori_loop(0, _NCHUNK, p1, (z, z, z))

    n = jnp.float32(_N_TOTAL)
    total_s = jnp.sum(s)
    total_ss = jnp.sum(ss)
    max_abs = jnp.max(m)
    var = (total_ss - total_s * total_s / n) / (n - 1.0)
    inv_sig = 1.0 / (jnp.sqrt(var) + 1e-6)
    k_e = _ENERGY_W / (max_abs * max_abs + 1e-6)
    b2 = inv_sig * _LOG2E
    a2 = 2.0 * _LOG2E

    def p2(g, _):
        x = x_vmem[pl.ds(g * _CHO, _CHO), :]
        e = jnp.exp2(a2 - jnp.abs(x) * b2)
        x_vmem[pl.ds(g * _CHO, _CHO), :] = (
            1.0 / (2.0 + 2.0 * e) + (k_e * (x * x) + _RECENCY_C)
        )
        pltpu.make_async_copy(
            x_vmem.at[pl.ds(g * _CHO, _CHO)],
            o_hbm.at[pl.ds(g * _CHO, _CHO)],
            sem_out.at[g],
        ).start()
        return 0

    jax.lax.fori_loop(0, _NCHUNKO, p2, 0)

    def drain(g, _):
        pltpu.make_async_copy(
            x_vmem.at[pl.ds(g * _CHO, _CHO)],
            o_hbm.at[pl.ds(g * _CHO, _CHO)],
            sem_out.at[g],
        ).wait()
        return 0

    jax.lax.fori_loop(0, _NCHUNKO, drain, 0)


def kernel(weights):
    x = weights.reshape(_ROWS, 128)
    out = pl.pallas_call(
        _fused_kernel,
        in_specs=[pl.BlockSpec(memory_space=pl.ANY)],
        out_specs=pl.BlockSpec(memory_space=pl.ANY),
        out_shape=jax.ShapeDtypeStruct((_ROWS, 128), jnp.float32),
        scratch_shapes=[
            pltpu.VMEM((_ROWS, 128), jnp.float32),
            pltpu.SemaphoreType.DMA((_NCHUNK,)),
            pltpu.SemaphoreType.DMA((_NCHUNKO,)),
        ],
    )(x)
    return out.reshape(weights.shape)
